# baseline (speedup 1.0000x reference)
import functools

import jax
import jax.numpy as jnp
from jax.experimental import pallas as pl

N = 100000
D = 128
NF = 64
NK = 32
B = 512
OUT = 128
TN = 512
KB = NK // 2


def _leaky(v):
    return jnp.where(v >= 0, v, 0.2 * v)


def _bdot(a, b):
    return jnp.dot(a.astype(jnp.bfloat16), b.astype(jnp.bfloat16),
                   preferred_element_type=jnp.float32)


def _split3(a):
    a0 = a.astype(jnp.bfloat16)
    r1 = a - a0.astype(jnp.float32)
    a1 = r1.astype(jnp.bfloat16)
    r2 = r1 - a1.astype(jnp.float32)
    a2 = r2.astype(jnp.bfloat16)
    return a0, a1, a2


def _big(v):
    return jnp.broadcast_to(v[None], (KB,) + v.shape).reshape(
        KB * v.shape[0], v.shape[1])


def _feat_kernel(x_ref, pos_ref, batch_ref, wx1t_ref, bx1_ref, cx_ref,
                 cy_ref, cz_ref, inv_ref, wx2_ref, wpos_ref, bias_ref,
                 pooled_ref):
    i = pl.program_id(0)

    x1 = _bdot(x_ref[...], wx1t_ref[...]) + bx1_ref[...]
    x1big = _big(jnp.concatenate([x1, x1], axis=1))

    px = _big(pos_ref[:, 0:1])
    py = _big(pos_ref[:, 1:2])
    pz = _big(pos_ref[:, 2:3])
    dx = px - cx_ref[...]
    dy = py - cy_ref[...]
    dz = pz - cz_ref[...]
    d2 = dx * dx + dy * dy + dz * dz
    rbf = jnp.exp(-jnp.sqrt(d2) * inv_ref[...])

    mul = rbf * x1big
    t = mul + rbf
    tb = t.astype(jnp.bfloat16).astype(jnp.float32)
    rb = rbf.astype(jnp.bfloat16).astype(jnp.float32)
    u = tb * wx2_ref[...] + rb * wpos_ref[...]

    s = jnp.sum(u.reshape(KB, TN, 2 * NF), axis=0)
    g = _leaky(s[:, :NF] + s[:, NF:] + bias_ref[...])

    oneh = (batch_ref[...] == jax.lax.broadcasted_iota(
        jnp.int32, (TN, B), 1)).astype(jnp.bfloat16)
    g0, g1, g2 = _split3(g)
    dn = (((0,), (0,)), ((), ()))
    part = (jax.lax.dot_general(oneh, g0, dn, preferred_element_type=jnp.float32)
            + jax.lax.dot_general(oneh, g1, dn, preferred_element_type=jnp.float32)
            + jax.lax.dot_general(oneh, g2, dn, preferred_element_type=jnp.float32))

    @pl.when(i == 0)
    def _init():
        pooled_ref[...] = jnp.zeros_like(pooled_ref)

    pooled_ref[...] += part


def _mlp_kernel(p_ref, w1t_ref, b1_ref, g1_ref, be1_ref, w2t_ref, b2_ref,
                g2_ref, be2_ref, out_ref):
    h = _bdot(p_ref[...], w1t_ref[...]) + b1_ref[...]
    m = jnp.mean(h, axis=0, keepdims=True)
    v = jnp.mean((h - m) ** 2, axis=0, keepdims=True)
    h = (h - m) / jnp.sqrt(v + 1e-5) * g1_ref[...] + be1_ref[...]
    h = _leaky(h)
    h = _bdot(h, w2t_ref[...]) + b2_ref[...]
    m = jnp.mean(h, axis=0, keepdims=True)
    v = jnp.mean((h - m) ** 2, axis=0, keepdims=True)
    out_ref[...] = (h - m) / jnp.sqrt(v + 1e-5) * g2_ref[...] + be2_ref[...]


def _layout_big(a_ik):
    a = a_ik.reshape(NF, KB, 2)
    a = jnp.transpose(a, (1, 2, 0))
    a = a.reshape(KB, 1, 2 * NF)
    return jnp.broadcast_to(a, (KB, TN, 2 * NF)).reshape(KB * TN, 2 * NF)


@jax.jit
def _run(x, pos, batch, centers, widths, Wx1, bx1, Wx2, bx2, Wpos, bpos,
         W1, b1, g1, be1, W2, b2, g2, be2):
    n = x.shape[0]
    n_pad = ((n + TN - 1) // TN) * TN
    grid = n_pad // TN

    x_p = jnp.pad(x, ((0, n_pad - n), (0, 0)))
    pos_p = jnp.pad(pos, ((0, n_pad - n), (0, 0)))
    batch_p = jnp.pad(batch, (0, n_pad - n), constant_values=B)[:, None]

    cxb = _layout_big(centers[:, :, 0])
    cyb = _layout_big(centers[:, :, 1])
    czb = _layout_big(centers[:, :, 2])
    invb = _layout_big(1.0 / (2.0 * widths ** 2))
    wx2b = _layout_big(Wx2).astype(jnp.bfloat16).astype(jnp.float32)
    wposb = _layout_big(Wpos).astype(jnp.bfloat16).astype(jnp.float32)
    bias = (bx2 + bpos)[None, :]

    big = (KB * TN, 2 * NF)
    pooled = pl.pallas_call(
        _feat_kernel,
        grid=(grid,),
        in_specs=[
            pl.BlockSpec((TN, D), lambda i: (i, 0)),
            pl.BlockSpec((TN, 3), lambda i: (i, 0)),
            pl.BlockSpec((TN, 1), lambda i: (i, 0)),
            pl.BlockSpec((D, NF), lambda i: (0, 0)),
            pl.BlockSpec((1, NF), lambda i: (0, 0)),
            pl.BlockSpec(big, lambda i: (0, 0)),
            pl.BlockSpec(big, lambda i: (0, 0)),
            pl.BlockSpec(big, lambda i: (0, 0)),
            pl.BlockSpec(big, lambda i: (0, 0)),
            pl.BlockSpec(big, lambda i: (0, 0)),
            pl.BlockSpec(big, lambda i: (0, 0)),
            pl.BlockSpec((1, NF), lambda i: (0, 0)),
        ],
        out_specs=pl.BlockSpec((B, NF), lambda i: (0, 0)),
        out_shape=jax.ShapeDtypeStruct((B, NF), jnp.float32),
    )(x_p, pos_p, batch_p, Wx1.T, bx1[None, :], cxb, cyb, czb, invb,
      wx2b, wposb, bias)

    out = pl.pallas_call(
        _mlp_kernel,
        out_shape=jax.ShapeDtypeStruct((B, OUT), jnp.float32),
    )(pooled, W1.T, b1[None, :], g1[None, :], be1[None, :],
      W2.T, b2[None, :], g2[None, :], be2[None, :])
    return out


def kernel(x, pos, batch, centers, widths, Wx1, bx1, Wx2, bx2, Wpos, bpos,
           W1, b1, g1, be1, W2, b2, g2, be2):
    return _run(x, pos, batch, centers, widths, Wx1, bx1, Wx2, bx2,
                Wpos, bpos, W1, b1, g1, be1, W2, b2, g2, be2)


# speedup vs baseline: 1.1294x; 1.1294x over previous
import functools

import jax
import jax.numpy as jnp
from jax.experimental import pallas as pl

N = 100000
D = 128
NF = 64
NK = 32
B = 512
OUT = 128
TN = 1024


def _leaky(v):
    return jnp.where(v >= 0, v, 0.2 * v)


def _bdot(a, b):
    return jnp.dot(a.astype(jnp.bfloat16), b.astype(jnp.bfloat16),
                   preferred_element_type=jnp.float32)


def _split3(a):
    a0 = a.astype(jnp.bfloat16)
    r1 = a - a0.astype(jnp.float32)
    a1 = r1.astype(jnp.bfloat16)
    r2 = r1 - a1.astype(jnp.float32)
    a2 = r2.astype(jnp.bfloat16)
    return a0, a1, a2


def _feat_kernel(x_ref, pos_ref, batch_ref, wx1w_ref, bx1w_ref, cx_ref,
                 cy_ref, cz_ref, inv_ref, mx2_ref, mpos_ref, bias_ref,
                 pooled_ref):
    i = pl.program_id(0)

    x1w = _bdot(x_ref[...], wx1w_ref[...]) + bx1w_ref[...]

    px = pos_ref[:, 0:1]
    py = pos_ref[:, 1:2]
    pz = pos_ref[:, 2:3]
    dx = px - cx_ref[...]
    dy = py - cy_ref[...]
    dz = pz - cz_ref[...]
    d2 = dx * dx + dy * dy + dz * dz
    rbf = jnp.exp(-jnp.sqrt(d2) * inv_ref[...])

    mul = rbf * x1w
    t = mul + rbf
    x2 = _bdot(t, mx2_ref[...])
    p1 = _bdot(rbf, mpos_ref[...])
    g = _leaky(x2 + p1 + bias_ref[...])

    oneh = (batch_ref[...] == jax.lax.broadcasted_iota(
        jnp.int32, (TN, B), 1)).astype(jnp.bfloat16)
    g0, g1, g2 = _split3(g)
    dn = (((0,), (0,)), ((), ()))
    part = (jax.lax.dot_general(oneh, g0, dn, preferred_element_type=jnp.float32)
            + jax.lax.dot_general(oneh, g1, dn, preferred_element_type=jnp.float32)
            + jax.lax.dot_general(oneh, g2, dn, preferred_element_type=jnp.float32))

    @pl.when(i == 0)
    def _init():
        pooled_ref[...] = jnp.zeros_like(pooled_ref)

    pooled_ref[...] += part


def _mlp_kernel(p_ref, w1t_ref, b1_ref, g1_ref, be1_ref, w2t_ref, b2_ref,
                g2_ref, be2_ref, out_ref):
    h = _bdot(p_ref[...], w1t_ref[...]) + b1_ref[...]
    m = jnp.mean(h, axis=0, keepdims=True)
    v = jnp.mean((h - m) ** 2, axis=0, keepdims=True)
    h = (h - m) / jnp.sqrt(v + 1e-5) * g1_ref[...] + be1_ref[...]
    h = _leaky(h)
    h = _bdot(h, w2t_ref[...]) + b2_ref[...]
    m = jnp.mean(h, axis=0, keepdims=True)
    v = jnp.mean((h - m) ** 2, axis=0, keepdims=True)
    out_ref[...] = (h - m) / jnp.sqrt(v + 1e-5) * g2_ref[...] + be2_ref[...]


@jax.jit
def _run(x, pos, batch, centers, widths, Wx1, bx1, Wx2, bx2, Wpos, bpos,
         W1, b1, g1, be1, W2, b2, g2, be2):
    n = x.shape[0]
    n_pad = ((n + TN - 1) // TN) * TN
    grid = n_pad // TN

    x_p = jnp.pad(x, ((0, n_pad - n), (0, 0)))
    pos_p = jnp.pad(pos, ((0, n_pad - n), (0, 0)))
    batch_p = jnp.pad(batch, (0, n_pad - n), constant_values=B)[:, None]

    c = centers.reshape(NF * NK, 3)
    cx = c[:, 0][None, :]
    cy = c[:, 1][None, :]
    cz = c[:, 2][None, :]
    inv = (1.0 / (2.0 * widths.reshape(NF * NK) ** 2))[None, :]

    wx1w = jnp.repeat(Wx1.T, NK, axis=1)
    bx1w = jnp.repeat(bx1, NK)[None, :]

    eye = jnp.eye(NF, dtype=jnp.float32)
    mx2 = (Wx2[:, :, None] * eye[:, None, :]).reshape(NF * NK, NF)
    mpos = (Wpos[:, :, None] * eye[:, None, :]).reshape(NF * NK, NF)
    bias = (bx2 + bpos)[None, :]

    pooled = pl.pallas_call(
        _feat_kernel,
        grid=(grid,),
        in_specs=[
            pl.BlockSpec((TN, D), lambda i: (i, 0)),
            pl.BlockSpec((TN, 3), lambda i: (i, 0)),
            pl.BlockSpec((TN, 1), lambda i: (i, 0)),
            pl.BlockSpec((D, NF * NK), lambda i: (0, 0)),
            pl.BlockSpec((1, NF * NK), lambda i: (0, 0)),
            pl.BlockSpec((1, NF * NK), lambda i: (0, 0)),
            pl.BlockSpec((1, NF * NK), lambda i: (0, 0)),
            pl.BlockSpec((1, NF * NK), lambda i: (0, 0)),
            pl.BlockSpec((1, NF * NK), lambda i: (0, 0)),
            pl.BlockSpec((NF * NK, NF), lambda i: (0, 0)),
            pl.BlockSpec((NF * NK, NF), lambda i: (0, 0)),
            pl.BlockSpec((1, NF), lambda i: (0, 0)),
        ],
        out_specs=pl.BlockSpec((B, NF), lambda i: (0, 0)),
        out_shape=jax.ShapeDtypeStruct((B, NF), jnp.float32),
    )(x_p, pos_p, batch_p, wx1w, bx1w, cx, cy, cz, inv, mx2, mpos, bias)

    out = pl.pallas_call(
        _mlp_kernel,
        out_shape=jax.ShapeDtypeStruct((B, OUT), jnp.float32),
    )(pooled, W1.T, b1[None, :], g1[None, :], be1[None, :],
      W2.T, b2[None, :], g2[None, :], be2[None, :])
    return out


def kernel(x, pos, batch, centers, widths, Wx1, bx1, Wx2, bx2, Wpos, bpos,
           W1, b1, g1, be1, W2, b2, g2, be2):
    return _run(x, pos, batch, centers, widths, Wx1, bx1, Wx2, bx2,
                Wpos, bpos, W1, b1, g1, be1, W2, b2, g2, be2)
